# baseline (speedup 1.0000x reference)
import functools

import jax
import jax.numpy as jnp
import numpy as np
from jax.experimental import pallas as pl
from jax.experimental.pallas import tpu as pltpu

_BINS = 10
_EDGES = [np.float32(np.float64(i) / _BINS) for i in range(_BINS + 1)]
_EDGES[-1] = np.float32(1.0 + 1e-06)
_CHUNK = 8
_BIG = np.float32(8192.0)
_INV_BIG = np.float32(1.0 / 8192.0)

_DL = np.zeros((128, 128), np.float32)
_DR = np.zeros((128, 128), np.float32)
_T0L = np.zeros((128, 128), np.float32)
_T0R = np.zeros((128, 128), np.float32)
_T1L = np.zeros((128, 128), np.float32)
_T1R = np.zeros((128, 128), np.float32)
for _j in range(64):
    _DL[2 * _j, _j] = 1.0
    _DL[2 * _j + 1, _j] = -1.0
    _DR[2 * _j, 64 + _j] = 1.0
    _DR[2 * _j + 1, 64 + _j] = -1.0
    _T0L[2 * _j, _j] = 1.0
    _T0R[2 * _j, 64 + _j] = 1.0
    _T1L[2 * _j + 1, _j] = 1.0
    _T1R[2 * _j + 1, 64 + _j] = 1.0

_DOT_DIMS = (((1,), (0,)), ((), ()))


def _dot(a, w):
    return jax.lax.dot_general(a, w, _DOT_DIMS,
                               preferred_element_type=jnp.float32)


def _ghmc_kernel(x_ref, t_ref, dl_ref, dr_ref, t0l_ref, t0r_ref, t1l_ref,
                 t1r_ref, o_ref, acc_ref, d_buf, t0_buf, t1_buf,
                 *, n_rows, n_steps, tot):
    step = pl.program_id(0)

    @pl.when(step == 0)
    def _init():
        acc_ref[...] = jnp.zeros_like(acc_ref)

    xb = x_ref[...]
    tb = t_ref[...]
    xh = xb.astype(jnp.bfloat16)
    xl = (xb - xh.astype(jnp.float32)).astype(jnp.bfloat16)
    th = tb.astype(jnp.bfloat16)
    tl = (tb - th.astype(jnp.float32)).astype(jnp.bfloat16)
    dl, dr = dl_ref[...], dr_ref[...]
    t0l, t0r = t0l_ref[...], t0r_ref[...]
    t1l, t1r = t1l_ref[...], t1r_ref[...]

    n_chunks = n_rows // (2 * _CHUNK)
    for k in range(n_chunks):
        o = pl.ds(k * _CHUNK, _CHUNK)
        d_buf[o, :] = (_dot(xh[2 * k * _CHUNK:(2 * k + 1) * _CHUNK], dl)
                       + _dot(xl[2 * k * _CHUNK:(2 * k + 1) * _CHUNK], dl)
                       + _dot(xh[(2 * k + 1) * _CHUNK:(2 * k + 2) * _CHUNK], dr)
                       + _dot(xl[(2 * k + 1) * _CHUNK:(2 * k + 2) * _CHUNK], dr))
        ta_h = th[2 * k * _CHUNK:(2 * k + 1) * _CHUNK]
        ta_l = tl[2 * k * _CHUNK:(2 * k + 1) * _CHUNK]
        tb_h = th[(2 * k + 1) * _CHUNK:(2 * k + 2) * _CHUNK]
        tb_l = tl[(2 * k + 1) * _CHUNK:(2 * k + 2) * _CHUNK]
        t0_buf[o, :] = (_dot(ta_h, t0l) + _dot(ta_l, t0l)
                        + _dot(tb_h, t0r) + _dot(tb_l, t0r))
        t1_buf[o, :] = (_dot(ta_h, t1l) + _dot(ta_l, t1l)
                        + _dot(tb_h, t1r) + _dot(tb_l, t1r))

    carry = [jnp.zeros((_CHUNK, 128), jnp.float32) for _ in range(_BINS + 1)]
    for i in range(n_chunks):
        d = d_buf[pl.ds(i * _CHUNK, _CHUNK), :]
        t0 = t0_buf[pl.ds(i * _CHUNK, _CHUNK), :]
        t1 = t1_buf[pl.ds(i * _CHUNK, _CHUNK), :]

        ad = jnp.abs(d)
        q = jnp.exp(-ad)
        u = 1.0 + q
        r = 1.0 / u
        p0 = jnp.where(d >= 0.0, r, 1.0 - r)
        p1 = 1.0 - p0
        g = jnp.abs(p0 - t0) * t0 + jnp.abs(p1 - t1) * t1
        z = jnp.where(t0 >= t1, -d, d)
        ce = jnp.maximum(z, 0.0) + jnp.log(u)
        w = ce + _BIG

        for j in range(_BINS):
            m = g >= _EDGES[j + 1]
            carry[j] = carry[j] + jnp.where(m, w, 0.0)
        carry[_BINS] = carry[_BINS] + ce

    for j in range(_BINS):
        cnt = jnp.floor(carry[j] * _INV_BIG)
        s = carry[j] - cnt * _BIG
        acc_ref[j] += cnt
        acc_ref[_BINS + 1 + j] += s
    acc_ref[_BINS] += carry[_BINS]

    @pl.when(step == n_steps - 1)
    def _fin():
        C = [jnp.sum(acc_ref[j]) for j in range(_BINS)]
        S = [jnp.sum(acc_ref[_BINS + j]) for j in range(_BINS + 1)]
        cnts = [tot - C[0]]
        sums = [S[0] - S[1]]
        for b in range(1, _BINS):
            cnts.append(C[b - 1] - C[b])
            sums.append(S[b] - S[b + 1])
        n = jnp.zeros((), jnp.float32)
        total = jnp.zeros((), jnp.float32)
        for b in range(_BINS):
            nonempty = cnts[b] > 0.0
            n = n + nonempty.astype(jnp.float32)
            contrib = sums[b] / (np.float32(0.1) * jnp.maximum(cnts[b], 1.0))
            total = total + jnp.where(nonempty, contrib, 0.0)
        o_ref[0, 0] = total / jnp.maximum(n, 1.0)


def kernel(input, target):
    n, c = input.shape
    assert c == 2
    m = 2 * n // 128
    x = input.reshape(m, 128)
    t = target.reshape(m, 128)
    rows = min(1024, m)
    steps = m // rows
    wspec = pl.BlockSpec((128, 128), lambda i: (0, 0))
    out = pl.pallas_call(
        functools.partial(
            _ghmc_kernel, n_rows=rows, n_steps=steps, tot=np.float32(n)
        ),
        grid=(steps,),
        in_specs=[
            pl.BlockSpec((rows, 128), lambda i: (i, 0)),
            pl.BlockSpec((rows, 128), lambda i: (i, 0)),
            wspec, wspec, wspec, wspec, wspec, wspec,
        ],
        out_specs=pl.BlockSpec(memory_space=pltpu.SMEM),
        out_shape=jax.ShapeDtypeStruct((1, 1), jnp.float32),
        scratch_shapes=[
            pltpu.VMEM((2 * _BINS + 1, 8, 128), jnp.float32),
            pltpu.VMEM((rows // 2, 128), jnp.float32),
            pltpu.VMEM((rows // 2, 128), jnp.float32),
            pltpu.VMEM((rows // 2, 128), jnp.float32),
        ],
    )(x, t,
      jnp.asarray(_DL, jnp.bfloat16), jnp.asarray(_DR, jnp.bfloat16),
      jnp.asarray(_T0L, jnp.bfloat16), jnp.asarray(_T0R, jnp.bfloat16),
      jnp.asarray(_T1L, jnp.bfloat16), jnp.asarray(_T1R, jnp.bfloat16))
    return out[0, 0]


# speedup vs baseline: 100.9941x; 100.9941x over previous
import functools

import jax
import jax.numpy as jnp
import numpy as np
from jax.experimental import pallas as pl
from jax.experimental.pallas import tpu as pltpu

_BINS = 10
_EDGES = [np.float32(np.float64(i) / _BINS) for i in range(_BINS + 1)]
_EDGES[-1] = np.float32(1.0 + 1e-06)
_CHUNK = 8
_BIG = np.float32(8192.0)
_INV_BIG = np.float32(1.0 / 8192.0)


def _ghmc_kernel(x0_ref, x1_ref, t0_ref, t1_ref, o_ref, acc_ref,
                 *, n_rows, n_steps, tot):
    step = pl.program_id(0)

    @pl.when(step == 0)
    def _init():
        acc_ref[...] = jnp.zeros_like(acc_ref)

    n_chunks = n_rows // _CHUNK
    carry = [jnp.zeros((_CHUNK, 128), jnp.float32) for _ in range(_BINS + 1)]
    for i in range(n_chunks):
        sl = pl.ds(i * _CHUNK, _CHUNK)
        x0 = x0_ref[0, sl, :]
        x1 = x1_ref[0, sl, :]
        t0 = t0_ref[0, sl, :]
        t1 = t1_ref[0, sl, :]

        d = x0 - x1
        ad = jnp.abs(d)
        q = jnp.exp(-ad)
        u = 1.0 + q
        r = 1.0 / u
        p0 = jnp.where(d >= 0.0, r, 1.0 - r)
        p1 = 1.0 - p0
        g = jnp.abs(p0 - t0) * t0 + jnp.abs(p1 - t1) * t1
        z = jnp.where(t0 >= t1, -d, d)
        ce = jnp.maximum(z, 0.0) + jnp.log(u)
        w = ce + _BIG

        for j in range(_BINS):
            m = g >= _EDGES[j + 1]
            carry[j] = carry[j] + jnp.where(m, w, 0.0)
        carry[_BINS] = carry[_BINS] + ce

    for j in range(_BINS):
        cnt = jnp.floor(carry[j] * _INV_BIG)
        s = carry[j] - cnt * _BIG
        acc_ref[j] += cnt
        acc_ref[_BINS + 1 + j] += s
    acc_ref[_BINS] += carry[_BINS]

    @pl.when(step == n_steps - 1)
    def _fin():
        C = [jnp.sum(acc_ref[j]) for j in range(_BINS)]
        S = [jnp.sum(acc_ref[_BINS + j]) for j in range(_BINS + 1)]
        cnts = [tot - C[0]]
        sums = [S[0] - S[1]]
        for b in range(1, _BINS):
            cnts.append(C[b - 1] - C[b])
            sums.append(S[b] - S[b + 1])
        n = jnp.zeros((), jnp.float32)
        total = jnp.zeros((), jnp.float32)
        for b in range(_BINS):
            nonempty = cnts[b] > 0.0
            n = n + nonempty.astype(jnp.float32)
            contrib = sums[b] / (np.float32(0.1) * jnp.maximum(cnts[b], 1.0))
            total = total + jnp.where(nonempty, contrib, 0.0)
        o_ref[0, 0] = total / jnp.maximum(n, 1.0)


def kernel(input, target):
    n, c = input.shape
    assert c == 2
    m = n // 128
    xt = jnp.swapaxes(input, 0, 1).reshape(2, m, 128)
    tt = jnp.swapaxes(target, 0, 1).reshape(2, m, 128)
    rows = min(512, m)
    steps = m // rows
    spec0 = pl.BlockSpec((1, rows, 128), lambda i: (0, i, 0))
    spec1 = pl.BlockSpec((1, rows, 128), lambda i: (1, i, 0))
    out = pl.pallas_call(
        functools.partial(
            _ghmc_kernel, n_rows=rows, n_steps=steps, tot=np.float32(n)
        ),
        grid=(steps,),
        in_specs=[spec0, spec1, spec0, spec1],
        out_specs=pl.BlockSpec(memory_space=pltpu.SMEM),
        out_shape=jax.ShapeDtypeStruct((1, 1), jnp.float32),
        scratch_shapes=[pltpu.VMEM((2 * _BINS + 1, 8, 128), jnp.float32)],
    )(xt, xt, tt, tt)
    return out[0, 0]


# speedup vs baseline: 127.0185x; 1.2577x over previous
import functools

import jax
import jax.numpy as jnp
import numpy as np
from jax.experimental import pallas as pl
from jax.experimental.pallas import tpu as pltpu

_BINS = 10
_EDGES = [np.float32(np.float64(i) / _BINS) for i in range(_BINS + 1)]
_EDGES[-1] = np.float32(1.0 + 1e-06)
_CHUNK = 8
_BIG = np.float32(8192.0)
_INV_BIG = np.float32(1.0 / 8192.0)


def _ghmc_kernel(x_ref, t_ref, o_ref, acc_ref, *, n_rows, n_steps, tot):
    step = pl.program_id(0)

    @pl.when(step == 0)
    def _init():
        acc_ref[...] = jnp.zeros_like(acc_ref)

    n_chunks = n_rows // _CHUNK
    carry = [jnp.zeros((_CHUNK, 128), jnp.float32) for _ in range(_BINS + 1)]
    for i in range(n_chunks):
        sl = pl.ds(i * _CHUNK, _CHUNK)
        x0 = x_ref[0, sl, :]
        x1 = x_ref[1, sl, :]
        t0 = t_ref[0, sl, :]
        t1 = t_ref[1, sl, :]

        d = x0 - x1
        ad = jnp.abs(d)
        q = jnp.exp(-ad)
        u = 1.0 + q
        r = 1.0 / u
        p0 = jnp.where(d >= 0.0, r, 1.0 - r)
        p1 = 1.0 - p0
        g = jnp.abs(p0 - t0) * t0 + jnp.abs(p1 - t1) * t1
        z = jnp.where(t0 >= t1, -d, d)
        ce = jnp.maximum(z, 0.0) + jnp.log(u)
        w = ce + _BIG

        for j in range(_BINS):
            m = g >= _EDGES[j + 1]
            carry[j] = carry[j] + jnp.where(m, w, 0.0)
        carry[_BINS] = carry[_BINS] + ce

    for j in range(_BINS):
        cnt = jnp.floor(carry[j] * _INV_BIG)
        s = carry[j] - cnt * _BIG
        acc_ref[j] += cnt
        acc_ref[_BINS + 1 + j] += s
    acc_ref[_BINS] += carry[_BINS]

    @pl.when(step == n_steps - 1)
    def _fin():
        C = [jnp.sum(acc_ref[j]) for j in range(_BINS)]
        S = [jnp.sum(acc_ref[_BINS + j]) for j in range(_BINS + 1)]
        cnts = [tot - C[0]]
        sums = [S[0] - S[1]]
        for b in range(1, _BINS):
            cnts.append(C[b - 1] - C[b])
            sums.append(S[b] - S[b + 1])
        n = jnp.zeros((), jnp.float32)
        total = jnp.zeros((), jnp.float32)
        for b in range(_BINS):
            nonempty = cnts[b] > 0.0
            n = n + nonempty.astype(jnp.float32)
            contrib = sums[b] / (np.float32(0.1) * jnp.maximum(cnts[b], 1.0))
            total = total + jnp.where(nonempty, contrib, 0.0)
        o_ref[0, 0] = total / jnp.maximum(n, 1.0)


def kernel(input, target):
    n, c = input.shape
    assert c == 2
    m = n // 128
    xt = jnp.swapaxes(input, 0, 1).reshape(2, m, 128)
    tt = jnp.swapaxes(target, 0, 1).reshape(2, m, 128)
    rows = min(2048, m)
    steps = m // rows
    spec = pl.BlockSpec((2, rows, 128), lambda i: (0, i, 0))
    out = pl.pallas_call(
        functools.partial(
            _ghmc_kernel, n_rows=rows, n_steps=steps, tot=np.float32(n)
        ),
        grid=(steps,),
        in_specs=[spec, spec],
        out_specs=pl.BlockSpec(memory_space=pltpu.SMEM),
        out_shape=jax.ShapeDtypeStruct((1, 1), jnp.float32),
        scratch_shapes=[pltpu.VMEM((2 * _BINS + 1, 8, 128), jnp.float32)],
    )(xt, tt)
    return out[0, 0]
